# baseline (speedup 1.0000x reference)
import jax
import jax.numpy as jnp
from jax.experimental import pallas as pl


def _dgc_body(x_ref, w1_ref, b1_ref, w2_ref, b2_ref, w3_ref, b3_ref, o_ref):
    xt = x_ref[0]
    q = jnp.dot(xt, w1_ref[:], preferred_element_type=jnp.float32) + b1_ref[0]
    k = jnp.dot(xt, w2_ref[:], preferred_element_type=jnp.float32) + b2_ref[0]
    v = jnp.dot(xt, w3_ref[:], preferred_element_type=jnp.float32) + b3_ref[0]
    s = jax.lax.dot_general(q, k, (((1,), (1,)), ((), ())),
                            preferred_element_type=jnp.float32)
    m = jnp.maximum(jnp.max(s, axis=1, keepdims=True), 0.0)
    e = jnp.exp(jnp.maximum(s - m, -m))
    denom = jnp.sum(e, axis=1, keepdims=True)
    out = jnp.dot(e.astype(jnp.bfloat16), v.astype(jnp.bfloat16),
                  preferred_element_type=jnp.float32) / denom
    o_ref[0] = jnp.maximum(out, 0.0)


def kernel(x, W1, b1, W2, b2, W3, b3):
    B, N, T, D = x.shape
    H = W1.shape[1]
    xs = x.transpose(0, 2, 1, 3).reshape(B * T, N, D)
    out = pl.pallas_call(
        _dgc_body,
        grid=(B * T,),
        in_specs=[
            pl.BlockSpec((1, N, D), lambda i: (i, 0, 0)),
            pl.BlockSpec((D, H), lambda i: (0, 0)),
            pl.BlockSpec((1, H), lambda i: (0, 0)),
            pl.BlockSpec((D, H), lambda i: (0, 0)),
            pl.BlockSpec((1, H), lambda i: (0, 0)),
            pl.BlockSpec((D, H), lambda i: (0, 0)),
            pl.BlockSpec((1, H), lambda i: (0, 0)),
        ],
        out_specs=pl.BlockSpec((1, N, H), lambda i: (i, 0, 0)),
        out_shape=jax.ShapeDtypeStruct((B * T, N, H), jnp.float32),
    )(xs, W1, b1.reshape(1, H), W2, b2.reshape(1, H), W3, b3.reshape(1, H))
    return out.reshape(B, T, N, H).transpose(0, 2, 1, 3)


# speedup vs baseline: 1.4331x; 1.4331x over previous
import jax
import jax.numpy as jnp
from jax.experimental import pallas as pl

_SLICES = 4


def _dgc_body(x_ref, w1_ref, b1_ref, w2_ref, b2_ref, w3_ref, b3_ref, o_ref):
    for j in range(_SLICES):
        xt = x_ref[j]
        q = jnp.dot(xt, w1_ref[:], preferred_element_type=jnp.float32) + b1_ref[0]
        k = jnp.dot(xt, w2_ref[:], preferred_element_type=jnp.float32) + b2_ref[0]
        v = jnp.dot(xt, w3_ref[:], preferred_element_type=jnp.float32) + b3_ref[0]
        s = jax.lax.dot_general(q, k, (((1,), (1,)), ((), ())),
                                preferred_element_type=jnp.float32)
        m = jnp.maximum(jnp.max(s, axis=1, keepdims=True), 0.0)
        e = jnp.exp(jnp.maximum(s - m, -m))
        denom = jnp.sum(e, axis=1, keepdims=True)
        out = jnp.dot(e, v, preferred_element_type=jnp.float32) / denom
        o_ref[j] = jnp.maximum(out, 0.0)


def kernel(x, W1, b1, W2, b2, W3, b3):
    B, N, T, D = x.shape
    H = W1.shape[1]
    xs = x.transpose(0, 2, 1, 3).reshape(B * T, N, D)
    grid = (B * T // _SLICES,)
    out = pl.pallas_call(
        _dgc_body,
        grid=grid,
        in_specs=[
            pl.BlockSpec((_SLICES, N, D), lambda i: (i, 0, 0)),
            pl.BlockSpec((D, H), lambda i: (0, 0)),
            pl.BlockSpec((1, H), lambda i: (0, 0)),
            pl.BlockSpec((D, H), lambda i: (0, 0)),
            pl.BlockSpec((1, H), lambda i: (0, 0)),
            pl.BlockSpec((D, H), lambda i: (0, 0)),
            pl.BlockSpec((1, H), lambda i: (0, 0)),
        ],
        out_specs=pl.BlockSpec((_SLICES, N, H), lambda i: (i, 0, 0)),
        out_shape=jax.ShapeDtypeStruct((B * T, N, H), jnp.float32),
    )(xs, W1, b1.reshape(1, H), W2, b2.reshape(1, H), W3, b3.reshape(1, H))
    return out.reshape(B, T, N, H).transpose(0, 2, 1, 3)
